# Initial kernel scaffold; baseline (speedup 1.0000x reference)
import jax
import jax.numpy as jnp
from jax.experimental import pallas as pl


def kernel(t, weight):
    raise NotImplementedError("write your pallas kernel here")


# speedup vs baseline: 2.4736x; 2.4736x over previous
import functools

import jax
import jax.numpy as jnp
from jax import lax
from jax.experimental import pallas as pl
from jax.experimental.pallas import tpu as pltpu
from jax.experimental.pallas import tpu_sc as plsc

SEQ = 8192
D = 768
BATCH = 4
TOTAL = BATCH * SEQ
NC, NS = 2, 16
NW = NC * NS
PER_W = TOTAL // NW
CH = 128
NCHUNK = PER_W // CH


def _build():
    mesh = plsc.VectorSubcoreMesh(core_axis_name="c", subcore_axis_name="s")

    @functools.partial(
        pl.kernel,
        mesh=mesh,
        out_type=jax.ShapeDtypeStruct((TOTAL, D), jnp.float32),
        scratch_types=[
            pltpu.VMEM((NCHUNK, CH), jnp.int32),
            pltpu.VMEM((CH, D), jnp.float32),
            pltpu.SemaphoreType.DMA,
        ],
    )
    def gather_kernel(idx_hbm, table_hbm, out_hbm, idx_v, rows_v, sem):
        wid = lax.axis_index("s") * NC + lax.axis_index("c")
        pltpu.sync_copy(idx_hbm.at[wid], idx_v)

        def body(j, carry):
            pltpu.async_copy(table_hbm.at[idx_v.at[j]], rows_v, sem).wait()
            pltpu.sync_copy(rows_v, out_hbm.at[pl.ds(wid * PER_W + j * CH, CH)])
            return carry

        lax.fori_loop(0, NCHUNK, body, 0)

    return gather_kernel


_gather = _build()


@jax.jit
def kernel(t, weight):
    idx = t.reshape(NW, NCHUNK, CH).astype(jnp.int32)
    out = _gather(idx, weight)
    return out.reshape(BATCH, SEQ, D)


# speedup vs baseline: 2.4841x; 1.0043x over previous
import functools

import jax
import jax.numpy as jnp
from jax import lax
from jax.experimental import pallas as pl
from jax.experimental.pallas import tpu as pltpu
from jax.experimental.pallas import tpu_sc as plsc

SEQ = 8192
D = 768
BATCH = 4
TOTAL = BATCH * SEQ
NC, NS = 2, 16
NW = NC * NS
PER_W = TOTAL // NW
CH = 64
NCHUNK = PER_W // CH


def _build():
    mesh = plsc.VectorSubcoreMesh(core_axis_name="c", subcore_axis_name="s")

    @functools.partial(
        pl.kernel,
        mesh=mesh,
        out_type=jax.ShapeDtypeStruct((TOTAL, D), jnp.float32),
        scratch_types=[
            pltpu.VMEM((NCHUNK, CH), jnp.int32),
            pltpu.VMEM((CH, D), jnp.float32),
            pltpu.VMEM((CH, D), jnp.float32),
            pltpu.SemaphoreType.DMA,
            pltpu.SemaphoreType.DMA,
            pltpu.SemaphoreType.DMA,
            pltpu.SemaphoreType.DMA,
        ],
    )
    def gather_kernel(idx_hbm, table_hbm, out_hbm, idx_v, rows0, rows1,
                      g0, g1, w0, w1):
        wid = lax.axis_index("s") * NC + lax.axis_index("c")
        base = wid * PER_W
        pltpu.sync_copy(idx_hbm.at[wid], idx_v)

        bufs = (rows0, rows1)
        gsems = (g0, g1)
        wsems = (w0, w1)
        gcp = [None, None]
        wcp = [None, None]
        for b in range(2):
            gcp[b] = pltpu.async_copy(table_hbm.at[idx_v.at[b]], bufs[b],
                                      gsems[b])
        for j in range(NCHUNK):
            b = j % 2
            gcp[b].wait()
            wcp[b] = pltpu.async_copy(
                bufs[b], out_hbm.at[pl.ds(base + j * CH, CH)], wsems[b])
            if j + 2 < NCHUNK:
                wcp[b].wait()
                gcp[b] = pltpu.async_copy(table_hbm.at[idx_v.at[j + 2]],
                                          bufs[b], gsems[b])
        wcp[0].wait()
        wcp[1].wait()

    return gather_kernel


_gather = _build()


@jax.jit
def kernel(t, weight):
    idx = t.reshape(NW, NCHUNK, CH).astype(jnp.int32)
    out = _gather(idx, weight)
    return out.reshape(BATCH, SEQ, D)
